# baseline (speedup 1.0000x reference)
import functools

import jax
import jax.numpy as jnp
from jax.experimental import pallas as pl
from jax.experimental.pallas import tpu as pltpu

TAU = 0.8


def _proj_kernel(x_ref, w_ref, b_ref, out_ref):
    z = jax.lax.dot_general(
        x_ref[...], w_ref[...],
        dimension_numbers=(((1,), (1,)), ((), ())),
        preferred_element_type=jnp.float32,
    ) + b_ref[...]
    z = jnp.where(z > 0, z, jnp.exp(z) - 1.0)
    norm = jnp.sqrt(jnp.sum(z * z, axis=1, keepdims=True))
    out_ref[...] = z / norm


def _sim_kernel(zn1_ref, zn2_ref, pos_ref, neg_ref, out_p_ref, out_n_ref):
    s = jax.lax.dot_general(
        zn1_ref[...], zn2_ref[...],
        dimension_numbers=(((1,), (1,)), ((), ())),
        preferred_element_type=jnp.float32,
    )
    s = jnp.exp(s * (1.0 / TAU))
    out_p_ref[0, 0, 0] = jnp.sum(s * pos_ref[...])
    out_n_ref[0, 0, 0] = jnp.sum(s * neg_ref[...])


def _largest_divisor(n, cap):
    for c in range(min(cap, n), 0, -1):
        if n % c == 0:
            return c
    return n


@jax.jit
def kernel(v1_embs, v2_embs, pos, neg, W, b):
    n, d = v1_embs.shape

    x = jnp.concatenate([v1_embs, v2_embs], axis=0)
    br = _largest_divisor(2 * n, 2000)
    zn = pl.pallas_call(
        _proj_kernel,
        grid=(2 * n // br,),
        in_specs=[
            pl.BlockSpec((br, d), lambda i: (i, 0)),
            pl.BlockSpec((d, d), lambda i: (0, 0)),
            pl.BlockSpec((1, d), lambda i: (0, 0)),
        ],
        out_specs=pl.BlockSpec((br, d), lambda i: (i, 0)),
        out_shape=jax.ShapeDtypeStruct((2 * n, d), jnp.float32),
    )(x, W, b.reshape(1, d))
    zn1, zn2 = zn[:n], zn[n:]

    bm = _largest_divisor(n, 200) if n % 8 == 0 else n
    ni = n // bm
    part_p, part_n = pl.pallas_call(
        _sim_kernel,
        grid=(ni,),
        in_specs=[
            pl.BlockSpec((bm, d), lambda i: (i, 0)),
            pl.BlockSpec((n, d), lambda i: (0, 0)),
            pl.BlockSpec((bm, n), lambda i: (i, 0)),
            pl.BlockSpec((bm, n), lambda i: (i, 0)),
        ],
        out_specs=[
            pl.BlockSpec((1, 1, 1), lambda i: (i, 0, 0), memory_space=pltpu.SMEM),
            pl.BlockSpec((1, 1, 1), lambda i: (i, 0, 0), memory_space=pltpu.SMEM),
        ],
        out_shape=[
            jax.ShapeDtypeStruct((ni, 1, 1), jnp.float32),
            jax.ShapeDtypeStruct((ni, 1, 1), jnp.float32),
        ],
        compiler_params=pltpu.CompilerParams(
            dimension_semantics=("arbitrary",),
        ),
    )(zn1, zn2, pos, neg)

    sum_p = jnp.sum(part_p)
    sum_n = jnp.sum(part_n)
    return jnp.log(sum_p + sum_n) - jnp.log(sum_p)


# speedup vs baseline: 1.0512x; 1.0512x over previous
import functools

import jax
import jax.numpy as jnp
from jax.experimental import pallas as pl
from jax.experimental.pallas import tpu as pltpu

TAU = 0.8


def _proj_kernel(x_ref, w_ref, b_ref, out_ref):
    z = jax.lax.dot_general(
        x_ref[...], w_ref[...],
        dimension_numbers=(((1,), (1,)), ((), ())),
        preferred_element_type=jnp.float32,
    ) + b_ref[...]
    z = jnp.where(z > 0, z, jnp.exp(z) - 1.0)
    norm = jnp.sqrt(jnp.sum(z * z, axis=1, keepdims=True))
    out_ref[...] = z / norm


def _sim_kernel(zn1_ref, zn2_ref, pos_ref, neg_ref, out_p_ref, out_n_ref):
    s = jax.lax.dot_general(
        zn1_ref[...], zn2_ref[...],
        dimension_numbers=(((1,), (1,)), ((), ())),
        preferred_element_type=jnp.float32,
    )
    s = jnp.exp(s * (1.0 / TAU))
    out_p_ref[0, 0, 0] = jnp.sum(s * pos_ref[...])
    out_n_ref[0, 0, 0] = jnp.sum(s * neg_ref[...])


def _largest_divisor(n, cap):
    for c in range(min(cap, n), 0, -1):
        if n % c == 0:
            return c
    return n


@jax.jit
def kernel(v1_embs, v2_embs, pos, neg, W, b):
    n, d = v1_embs.shape

    br = _largest_divisor(n, 2000)
    proj = pl.pallas_call(
        _proj_kernel,
        grid=(n // br,),
        in_specs=[
            pl.BlockSpec((br, d), lambda i: (i, 0)),
            pl.BlockSpec((d, d), lambda i: (0, 0)),
            pl.BlockSpec((1, d), lambda i: (0, 0)),
        ],
        out_specs=pl.BlockSpec((br, d), lambda i: (i, 0)),
        out_shape=jax.ShapeDtypeStruct((n, d), jnp.float32),
    )
    b2 = b.reshape(1, d)
    zn1 = proj(v1_embs, W, b2)
    zn2 = proj(v2_embs, W, b2)

    bm = _largest_divisor(n, 200) if n % 8 == 0 else n
    ni = n // bm
    part_p, part_n = pl.pallas_call(
        _sim_kernel,
        grid=(ni,),
        in_specs=[
            pl.BlockSpec((bm, d), lambda i: (i, 0)),
            pl.BlockSpec((n, d), lambda i: (0, 0)),
            pl.BlockSpec((bm, n), lambda i: (i, 0)),
            pl.BlockSpec((bm, n), lambda i: (i, 0)),
        ],
        out_specs=[
            pl.BlockSpec((1, 1, 1), lambda i: (i, 0, 0), memory_space=pltpu.SMEM),
            pl.BlockSpec((1, 1, 1), lambda i: (i, 0, 0), memory_space=pltpu.SMEM),
        ],
        out_shape=[
            jax.ShapeDtypeStruct((ni, 1, 1), jnp.float32),
            jax.ShapeDtypeStruct((ni, 1, 1), jnp.float32),
        ],
        compiler_params=pltpu.CompilerParams(
            dimension_semantics=("arbitrary",),
        ),
    )(zn1, zn2, pos, neg)

    sum_p = jnp.sum(part_p)
    sum_n = jnp.sum(part_n)
    return jnp.log(sum_p + sum_n) - jnp.log(sum_p)


# speedup vs baseline: 1.0532x; 1.0020x over previous
import functools

import jax
import jax.numpy as jnp
from jax.experimental import pallas as pl
from jax.experimental.pallas import tpu as pltpu

TAU = 0.8


def _proj_kernel(x_ref, w_ref, b_ref, out_ref):
    z = jax.lax.dot_general(
        x_ref[...], w_ref[...],
        dimension_numbers=(((1,), (1,)), ((), ())),
        preferred_element_type=jnp.float32,
    ) + b_ref[...]
    z = jnp.where(z > 0, z, jnp.exp(z) - 1.0)
    norm = jnp.sqrt(jnp.sum(z * z, axis=1, keepdims=True))
    out_ref[...] = z / norm


def _sim_kernel(zn1_ref, zn2_ref, pos_ref, neg_ref, out_p_ref, out_n_ref):
    s = jax.lax.dot_general(
        zn1_ref[...], zn2_ref[...],
        dimension_numbers=(((1,), (1,)), ((), ())),
        preferred_element_type=jnp.float32,
    )
    s = jnp.exp(s * (1.0 / TAU))
    out_p_ref[0, 0, 0] = jnp.sum(s * pos_ref[...])
    out_n_ref[0, 0, 0] = jnp.sum(s * neg_ref[...])


def _largest_divisor(n, cap):
    for c in range(min(cap, n), 0, -1):
        if n % c == 0:
            return c
    return n


@jax.jit
def kernel(v1_embs, v2_embs, pos, neg, W, b):
    n, d = v1_embs.shape

    br = _largest_divisor(n, 2000)
    proj = pl.pallas_call(
        _proj_kernel,
        grid=(n // br,),
        in_specs=[
            pl.BlockSpec((br, d), lambda i: (i, 0)),
            pl.BlockSpec((d, d), lambda i: (0, 0)),
            pl.BlockSpec((1, d), lambda i: (0, 0)),
        ],
        out_specs=pl.BlockSpec((br, d), lambda i: (i, 0)),
        out_shape=jax.ShapeDtypeStruct((n, d), jnp.float32),
    )
    b2 = b.reshape(1, d)
    zn1 = proj(v1_embs, W, b2)
    zn2 = proj(v2_embs, W, b2)

    bm = _largest_divisor(n, 200) if n % 8 == 0 else n
    ni = n // bm
    part_p, part_n = pl.pallas_call(
        _sim_kernel,
        grid=(ni,),
        in_specs=[
            pl.BlockSpec((bm, d), lambda i: (i, 0)),
            pl.BlockSpec((n, d), lambda i: (0, 0)),
            pl.BlockSpec((bm, n), lambda i: (i, 0)),
            pl.BlockSpec((bm, n), lambda i: (i, 0)),
        ],
        out_specs=[
            pl.BlockSpec((1, 1, 1), lambda i: (i, 0, 0), memory_space=pltpu.SMEM),
            pl.BlockSpec((1, 1, 1), lambda i: (i, 0, 0), memory_space=pltpu.SMEM),
        ],
        out_shape=[
            jax.ShapeDtypeStruct((ni, 1, 1), jnp.float32),
            jax.ShapeDtypeStruct((ni, 1, 1), jnp.float32),
        ],
        compiler_params=pltpu.CompilerParams(
            dimension_semantics=("parallel",),
        ),
    )(zn1, zn2, pos, neg)

    sum_p = jnp.sum(part_p)
    sum_n = jnp.sum(part_n)
    return jnp.log(sum_p + sum_n) - jnp.log(sum_p)


# speedup vs baseline: 1.0955x; 1.0401x over previous
import jax
import jax.numpy as jnp
from jax.experimental import pallas as pl
from jax.experimental.pallas import tpu as pltpu

TAU = 0.8


def _proj_normalize(x, w, b):
    z = jax.lax.dot_general(
        x, w,
        dimension_numbers=(((1,), (1,)), ((), ())),
        preferred_element_type=jnp.float32,
    ) + b
    z = jnp.where(z > 0, z, jnp.exp(z) - 1.0)
    norm = jnp.sqrt(jnp.sum(z * z, axis=1, keepdims=True))
    return z / norm


def _fused_kernel(v1_ref, v2_ref, w_ref, b_ref, pos_ref, neg_ref,
                  out_p_ref, out_n_ref, zn2_ref):
    @pl.when(pl.program_id(0) == 0)
    def _():
        zn2_ref[...] = _proj_normalize(v2_ref[...], w_ref[...], b_ref[...])

    zn1 = _proj_normalize(v1_ref[...], w_ref[...], b_ref[...])
    s = jax.lax.dot_general(
        zn1, zn2_ref[...],
        dimension_numbers=(((1,), (1,)), ((), ())),
        preferred_element_type=jnp.float32,
    )
    s = jnp.exp(s * (1.0 / TAU))
    out_p_ref[0, 0, 0] = jnp.sum(s * pos_ref[...])
    out_n_ref[0, 0, 0] = jnp.sum(s * neg_ref[...])


def _largest_divisor(n, cap):
    for c in range(min(cap, n), 0, -1):
        if n % c == 0:
            return c
    return n


@jax.jit
def kernel(v1_embs, v2_embs, pos, neg, W, b):
    n, d = v1_embs.shape
    bm = _largest_divisor(n, 200) if n % 8 == 0 else n
    ni = n // bm
    part_p, part_n = pl.pallas_call(
        _fused_kernel,
        grid=(ni,),
        in_specs=[
            pl.BlockSpec((bm, d), lambda i: (i, 0)),
            pl.BlockSpec((n, d), lambda i: (0, 0)),
            pl.BlockSpec((d, d), lambda i: (0, 0)),
            pl.BlockSpec((1, d), lambda i: (0, 0)),
            pl.BlockSpec((bm, n), lambda i: (i, 0)),
            pl.BlockSpec((bm, n), lambda i: (i, 0)),
        ],
        out_specs=[
            pl.BlockSpec((1, 1, 1), lambda i: (i, 0, 0), memory_space=pltpu.SMEM),
            pl.BlockSpec((1, 1, 1), lambda i: (i, 0, 0), memory_space=pltpu.SMEM),
        ],
        out_shape=[
            jax.ShapeDtypeStruct((ni, 1, 1), jnp.float32),
            jax.ShapeDtypeStruct((ni, 1, 1), jnp.float32),
        ],
        scratch_shapes=[pltpu.VMEM((n, d), jnp.float32)],
        compiler_params=pltpu.CompilerParams(
            dimension_semantics=("arbitrary",),
        ),
    )(v1_embs, v2_embs, W, b.reshape(1, d), pos, neg)

    sum_p = jnp.sum(part_p)
    sum_n = jnp.sum(part_n)
    return jnp.log(sum_p + sum_n) - jnp.log(sum_p)
